# Initial kernel scaffold; baseline (speedup 1.0000x reference)
import jax
import jax.numpy as jnp
from jax.experimental import pallas as pl


def kernel(batch, emb, rW1, rb1, rW2, rb2, Ws, bs, Wv, Wg, bg, cW1, cb1, cW2, cb2, cW3, cb3):
    raise NotImplementedError("write your pallas kernel here")


# speedup vs baseline: 9.2142x; 9.2142x over previous
import functools
import numpy as np
import jax
import jax.numpy as jnp
from jax.experimental import pallas as pl
from jax.experimental.pallas import tpu as pltpu

_B, _N = 8, 1024
_C = 64
_NUM_RBF = 32
_RH = 64
_L = 4
_K = 16
_CUTOFF = 5.0
_NCLS = 40


def _sigmoid(x):
    return 1.0 / (1.0 + jnp.exp(-x))


def _silu(x):
    return x * _sigmoid(x)


def _tfn_body(x_ref, emb_ref, rW1_ref, rb1_ref, rW2_ref, rb2_ref,
              Ws_ref, bs_ref, Wv_ref, Wg_ref, bg_ref,
              cW1_ref, cb1_ref, cW2_ref, cb2_ref, cW3_ref, cb3_ref,
              out_ref,
              P_ref, RU_ref, work_ref,
              s_ref, vx_ref, vy_ref, vz_ref,
              ms_ref, mvx_ref, mvy_ref, mvz_ref):
    f32 = jnp.float32
    x = x_ref[0]

    width = np.float32(_CUTOFF / (_NUM_RBF - 1))
    mu = jax.lax.broadcasted_iota(jnp.int32, (1, _NUM_RBF), 1).astype(f32) * width
    xn = jnp.sum(x * x, axis=1, keepdims=True)
    xxT = jax.lax.dot_general(x, x, (((1,), (1,)), ((), ())),
                              preferred_element_type=f32)
    d2 = xn + xn.T - 2.0 * xxT
    iota_j = jax.lax.broadcasted_iota(jnp.int32, (_N, _N), 1)
    iota_i = jax.lax.broadcasted_iota(jnp.int32, (_N, _N), 0)
    d2 = jnp.where(iota_i == iota_j, 1e9, d2)
    work_ref[...] = d2

    def _topk_body(k, _):
        work = work_ref[...]
        rmin = jnp.min(work, axis=1, keepdims=True)
        cand = jnp.where(work <= rmin, iota_j, jnp.int32(2**30))
        jmin = jnp.min(cand, axis=1, keepdims=True)
        onehot = (iota_j == jmin)
        work_ref[...] = jnp.where(onehot, 3e9, work)
        oh_f = jnp.where(onehot, 1.0, 0.0).astype(f32)
        P_ref[k] = oh_f.astype(jnp.bfloat16)
        xj = jnp.dot(oh_f, x, preferred_element_type=f32)
        rel = xj - x
        dist = jnp.sqrt(jnp.sum(rel * rel, axis=1, keepdims=True) + 1e-12)
        unit = rel / dist
        rbf = jnp.exp(-(((dist - mu) / width) ** 2))
        env = 0.5 * (jnp.cos(np.pi * jnp.clip(dist / _CUTOFF, 0.0, 1.0)) + 1.0)
        RU_ref[k, :, 0:_NUM_RBF] = rbf * env
        RU_ref[k, :, _NUM_RBF:_NUM_RBF + 3] = unit
        return 0

    jax.lax.fori_loop(0, _K, _topk_body, 0)

    s_ref[...] = jnp.broadcast_to(emb_ref[...], (_N, _C))
    vx_ref[...] = jnp.zeros((_N, _C), f32)
    vy_ref[...] = jnp.zeros((_N, _C), f32)
    vz_ref[...] = jnp.zeros((_N, _C), f32)

    for l in range(_L):
        table = jnp.concatenate(
            [s_ref[...], vx_ref[...], vy_ref[...], vz_ref[...]], axis=1)
        t_hi = table.astype(jnp.bfloat16)
        t_lo = (table - t_hi.astype(f32)).astype(jnp.bfloat16)

        ms_ref[...] = jnp.zeros((_N, _C), f32)
        mvx_ref[...] = jnp.zeros((_N, _C), f32)
        mvy_ref[...] = jnp.zeros((_N, _C), f32)
        mvz_ref[...] = jnp.zeros((_N, _C), f32)

        def _edge_body(k, _):
            Pk = P_ref[k]
            G = (jnp.dot(Pk, t_hi, preferred_element_type=f32) +
                 jnp.dot(Pk, t_lo, preferred_element_type=f32))
            R = RU_ref[k]
            rbf = R[:, 0:_NUM_RBF]
            ux = R[:, _NUM_RBF:_NUM_RBF + 1]
            uy = R[:, _NUM_RBF + 1:_NUM_RBF + 2]
            uz = R[:, _NUM_RBF + 2:_NUM_RBF + 3]

            h = _silu(jnp.dot(rbf, rW1_ref[l], preferred_element_type=f32)
                      + rb1_ref[l])
            w = jnp.dot(h, rW2_ref[l], preferred_element_type=f32) + rb2_ref[l]
            wss = w[:, 0:_C]
            wvs = w[:, _C:2 * _C]
            wsv = w[:, 2 * _C:3 * _C]
            wvv = w[:, 3 * _C:4 * _C]

            s_j = G[:, 0:_C]
            vjx = G[:, _C:2 * _C]
            vjy = G[:, 2 * _C:3 * _C]
            vjz = G[:, 3 * _C:4 * _C]
            vdotu = vjx * ux + vjy * uy + vjz * uz

            ms_ref[...] += wss * s_j + wvs * vdotu
            su = wsv * s_j
            mvx_ref[...] += wvv * vjx + su * ux
            mvy_ref[...] += wvv * vjy + su * uy
            mvz_ref[...] += wvv * vjz + su * uz
            return 0

        jax.lax.fori_loop(0, _K, _edge_body, 0)

        inv_k = f32(1.0 / _K)
        m_s = ms_ref[...] * inv_k
        s_cur = s_ref[...]
        s_upd = jnp.dot(m_s, Ws_ref[l], preferred_element_type=f32) + bs_ref[l]
        g = _sigmoid(jnp.dot(s_cur, Wg_ref[l], preferred_element_type=f32)
                     + bg_ref[l])
        gs = g[:, 0:_C]
        gv = g[:, _C:2 * _C]
        s_ref[...] = s_cur + gs * _silu(s_upd)
        Wvl = Wv_ref[l]
        vx_ref[...] += gv * jnp.dot(mvx_ref[...] * inv_k, Wvl,
                                    preferred_element_type=f32)
        vy_ref[...] += gv * jnp.dot(mvy_ref[...] * inv_k, Wvl,
                                    preferred_element_type=f32)
        vz_ref[...] += gv * jnp.dot(mvz_ref[...] * inv_k, Wvl,
                                    preferred_element_type=f32)

    vx, vy, vz = vx_ref[...], vy_ref[...], vz_ref[...]
    vn = jnp.sqrt(vx * vx + vy * vy + vz * vz + 1e-12)
    feat = jnp.concatenate([s_ref[...], vn], axis=1)
    pooled = jnp.mean(feat, axis=0, keepdims=True)
    h1 = jnp.maximum(jnp.dot(pooled, cW1_ref[...],
                             preferred_element_type=f32) + cb1_ref[...], 0.0)
    h2 = jnp.maximum(jnp.dot(h1, cW2_ref[...],
                             preferred_element_type=f32) + cb2_ref[...], 0.0)
    out_ref[0] = jnp.dot(h2, cW3_ref[...],
                         preferred_element_type=f32) + cb3_ref[...]


@jax.jit
def kernel(batch, emb, rW1, rb1, rW2, rb2, Ws, bs, Wv, Wg, bg,
           cW1, cb1, cW2, cb2, cW3, cb3):
    f32 = jnp.float32
    whole = lambda shape: pl.BlockSpec(shape, lambda b: (0,) * len(shape))
    in_specs = [
        pl.BlockSpec((1, _N, 3), lambda b: (b, 0, 0)),
        whole((1, _C)),
        whole((_L, _NUM_RBF, _RH)),
        whole((_L, 1, _RH)),
        whole((_L, _RH, 4 * _C)),
        whole((_L, 1, 4 * _C)),
        whole((_L, _C, _C)),
        whole((_L, 1, _C)),
        whole((_L, _C, _C)),
        whole((_L, _C, 2 * _C)),
        whole((_L, 1, 2 * _C)),
        whole((2 * _C, 128)),
        whole((1, 128)),
        whole((128, 64)),
        whole((1, 64)),
        whole((64, _NCLS)),
        whole((1, _NCLS)),
    ]
    out = pl.pallas_call(
        _tfn_body,
        grid=(_B,),
        in_specs=in_specs,
        out_specs=pl.BlockSpec((1, 1, _NCLS), lambda b: (b, 0, 0)),
        out_shape=jax.ShapeDtypeStruct((_B, 1, _NCLS), f32),
        scratch_shapes=[
            pltpu.VMEM((_K, _N, _N), jnp.bfloat16),
            pltpu.VMEM((_K, _N, 128), f32),
            pltpu.VMEM((_N, _N), f32),
            pltpu.VMEM((_N, _C), f32),
            pltpu.VMEM((_N, _C), f32),
            pltpu.VMEM((_N, _C), f32),
            pltpu.VMEM((_N, _C), f32),
            pltpu.VMEM((_N, _C), f32),
            pltpu.VMEM((_N, _C), f32),
            pltpu.VMEM((_N, _C), f32),
            pltpu.VMEM((_N, _C), f32),
        ],
    )(batch, emb.reshape(1, _C),
      rW1, rb1.reshape(_L, 1, _RH), rW2, rb2.reshape(_L, 1, 4 * _C),
      Ws, bs.reshape(_L, 1, _C), Wv, Wg, bg.reshape(_L, 1, 2 * _C),
      cW1, cb1.reshape(1, 128), cW2, cb2.reshape(1, 64),
      cW3, cb3.reshape(1, _NCLS))
    return out.reshape(_B, _NCLS)


# speedup vs baseline: 10.7911x; 1.1711x over previous
import functools
import numpy as np
import jax
import jax.numpy as jnp
from jax.experimental import pallas as pl
from jax.experimental.pallas import tpu as pltpu

_B, _N = 8, 1024
_C = 64
_NUM_RBF = 32
_RH = 64
_L = 4
_K = 16
_CUTOFF = 5.0
_NCLS = 40


def _sigmoid(x):
    return 1.0 / (1.0 + jnp.exp(-x))


def _silu(x):
    return x * _sigmoid(x)


def _tfn_body(x_ref, emb_ref, rW1_ref, rb1_ref, rW2_ref, rb2_ref,
              Ws_ref, bs_ref, Wv_ref, Wg_ref, bg_ref,
              cW1_ref, cb1_ref, cW2_ref, cb2_ref, cW3_ref, cb3_ref,
              out_ref,
              P_ref, RU_ref, work_ref,
              s_ref, vx_ref, vy_ref, vz_ref,
              ms_ref, mvx_ref, mvy_ref, mvz_ref):
    f32 = jnp.float32
    x = x_ref[0]

    width = np.float32(_CUTOFF / (_NUM_RBF - 1))
    mu = jax.lax.broadcasted_iota(jnp.int32, (1, _NUM_RBF), 1).astype(f32) * width
    xn = jnp.sum(x * x, axis=1, keepdims=True)
    xxT = jax.lax.dot_general(x, x, (((1,), (1,)), ((), ())),
                              preferred_element_type=f32)
    d2 = xn + xn.T - 2.0 * xxT
    iota_j = jax.lax.broadcasted_iota(jnp.int32, (_N, _N), 1)
    iota_i = jax.lax.broadcasted_iota(jnp.int32, (_N, _N), 0)
    d2 = jnp.where(iota_i == iota_j, 1e9, d2)
    work_ref[...] = d2

    def _topk_body(k, _):
        work = work_ref[...]
        rmin = jnp.min(work, axis=1, keepdims=True)
        cand = jnp.where(work <= rmin, iota_j, jnp.int32(2**30))
        jmin = jnp.min(cand, axis=1, keepdims=True)
        onehot = (iota_j == jmin)
        work_ref[...] = jnp.where(onehot, 3e9, work)
        oh_f = jnp.where(onehot, 1.0, 0.0).astype(f32)
        P_ref[k] = oh_f.astype(jnp.bfloat16)
        xj = jnp.dot(oh_f, x, preferred_element_type=f32)
        rel = xj - x
        dist = jnp.sqrt(jnp.sum(rel * rel, axis=1, keepdims=True) + 1e-12)
        unit = rel / dist
        rbf = jnp.exp(-(((dist - mu) / width) ** 2))
        env = 0.5 * (jnp.cos(np.pi * jnp.clip(dist / _CUTOFF, 0.0, 1.0)) + 1.0)
        RU_ref[k, :, 0:_NUM_RBF] = rbf * env
        RU_ref[k, :, _NUM_RBF:_NUM_RBF + 3] = unit
        return 0

    jax.lax.fori_loop(0, _K, _topk_body, 0)

    s_ref[...] = jnp.broadcast_to(emb_ref[...], (_N, _C))
    vx_ref[...] = jnp.zeros((_N, _C), f32)
    vy_ref[...] = jnp.zeros((_N, _C), f32)
    vz_ref[...] = jnp.zeros((_N, _C), f32)

    for l in range(_L):
        table = jnp.concatenate(
            [s_ref[...], vx_ref[...], vy_ref[...], vz_ref[...]], axis=1)
        t_hi = table.astype(jnp.bfloat16)

        ms_ref[...] = jnp.zeros((_N, _C), f32)
        mvx_ref[...] = jnp.zeros((_N, _C), f32)
        mvy_ref[...] = jnp.zeros((_N, _C), f32)
        mvz_ref[...] = jnp.zeros((_N, _C), f32)

        def _edge_one(k):
            Pk = P_ref[k]
            G = jnp.dot(Pk, t_hi, preferred_element_type=f32)
            R = RU_ref[k]
            rbf = R[:, 0:_NUM_RBF]
            ux = R[:, _NUM_RBF:_NUM_RBF + 1]
            uy = R[:, _NUM_RBF + 1:_NUM_RBF + 2]
            uz = R[:, _NUM_RBF + 2:_NUM_RBF + 3]

            h = _silu(jnp.dot(rbf, rW1_ref[l], preferred_element_type=f32)
                      + rb1_ref[l])
            w = jnp.dot(h, rW2_ref[l], preferred_element_type=f32) + rb2_ref[l]
            wss = w[:, 0:_C]
            wvs = w[:, _C:2 * _C]
            wsv = w[:, 2 * _C:3 * _C]
            wvv = w[:, 3 * _C:4 * _C]

            s_j = G[:, 0:_C]
            vjx = G[:, _C:2 * _C]
            vjy = G[:, 2 * _C:3 * _C]
            vjz = G[:, 3 * _C:4 * _C]
            vdotu = vjx * ux + vjy * uy + vjz * uz

            ms_ref[...] += wss * s_j + wvs * vdotu
            su = wsv * s_j
            mvx_ref[...] += wvv * vjx + su * ux
            mvy_ref[...] += wvv * vjy + su * uy
            mvz_ref[...] += wvv * vjz + su * uz

        def _edge_body(i, _):
            _edge_one(2 * i)
            _edge_one(2 * i + 1)
            return 0

        jax.lax.fori_loop(0, _K // 2, _edge_body, 0)

        inv_k = f32(1.0 / _K)
        m_s = ms_ref[...] * inv_k
        s_cur = s_ref[...]
        s_upd = jnp.dot(m_s, Ws_ref[l], preferred_element_type=f32) + bs_ref[l]
        g = _sigmoid(jnp.dot(s_cur, Wg_ref[l], preferred_element_type=f32)
                     + bg_ref[l])
        gs = g[:, 0:_C]
        gv = g[:, _C:2 * _C]
        s_ref[...] = s_cur + gs * _silu(s_upd)
        Wvl = Wv_ref[l]
        vx_ref[...] += gv * jnp.dot(mvx_ref[...] * inv_k, Wvl,
                                    preferred_element_type=f32)
        vy_ref[...] += gv * jnp.dot(mvy_ref[...] * inv_k, Wvl,
                                    preferred_element_type=f32)
        vz_ref[...] += gv * jnp.dot(mvz_ref[...] * inv_k, Wvl,
                                    preferred_element_type=f32)

    vx, vy, vz = vx_ref[...], vy_ref[...], vz_ref[...]
    vn = jnp.sqrt(vx * vx + vy * vy + vz * vz + 1e-12)
    feat = jnp.concatenate([s_ref[...], vn], axis=1)
    pooled = jnp.mean(feat, axis=0, keepdims=True)
    h1 = jnp.maximum(jnp.dot(pooled, cW1_ref[...],
                             preferred_element_type=f32) + cb1_ref[...], 0.0)
    h2 = jnp.maximum(jnp.dot(h1, cW2_ref[...],
                             preferred_element_type=f32) + cb2_ref[...], 0.0)
    out_ref[0] = jnp.dot(h2, cW3_ref[...],
                         preferred_element_type=f32) + cb3_ref[...]


@jax.jit
def kernel(batch, emb, rW1, rb1, rW2, rb2, Ws, bs, Wv, Wg, bg,
           cW1, cb1, cW2, cb2, cW3, cb3):
    f32 = jnp.float32
    whole = lambda shape: pl.BlockSpec(shape, lambda b: (0,) * len(shape))
    in_specs = [
        pl.BlockSpec((1, _N, 3), lambda b: (b, 0, 0)),
        whole((1, _C)),
        whole((_L, _NUM_RBF, _RH)),
        whole((_L, 1, _RH)),
        whole((_L, _RH, 4 * _C)),
        whole((_L, 1, 4 * _C)),
        whole((_L, _C, _C)),
        whole((_L, 1, _C)),
        whole((_L, _C, _C)),
        whole((_L, _C, 2 * _C)),
        whole((_L, 1, 2 * _C)),
        whole((2 * _C, 128)),
        whole((1, 128)),
        whole((128, 64)),
        whole((1, 64)),
        whole((64, _NCLS)),
        whole((1, _NCLS)),
    ]
    out = pl.pallas_call(
        _tfn_body,
        grid=(_B,),
        in_specs=in_specs,
        out_specs=pl.BlockSpec((1, 1, _NCLS), lambda b: (b, 0, 0)),
        out_shape=jax.ShapeDtypeStruct((_B, 1, _NCLS), f32),
        scratch_shapes=[
            pltpu.VMEM((_K, _N, _N), jnp.bfloat16),
            pltpu.VMEM((_K, _N, 128), f32),
            pltpu.VMEM((_N, _N), f32),
            pltpu.VMEM((_N, _C), f32),
            pltpu.VMEM((_N, _C), f32),
            pltpu.VMEM((_N, _C), f32),
            pltpu.VMEM((_N, _C), f32),
            pltpu.VMEM((_N, _C), f32),
            pltpu.VMEM((_N, _C), f32),
            pltpu.VMEM((_N, _C), f32),
            pltpu.VMEM((_N, _C), f32),
        ],
    )(batch, emb.reshape(1, _C),
      rW1, rb1.reshape(_L, 1, _RH), rW2, rb2.reshape(_L, 1, 4 * _C),
      Ws, bs.reshape(_L, 1, _C), Wv, Wg, bg.reshape(_L, 1, 2 * _C),
      cW1, cb1.reshape(1, 128), cW2, cb2.reshape(1, 64),
      cW3, cb3.reshape(1, _NCLS))
    return out.reshape(_B, _NCLS)


# speedup vs baseline: 11.8324x; 1.0965x over previous
import functools
import numpy as np
import jax
import jax.numpy as jnp
from jax.experimental import pallas as pl
from jax.experimental.pallas import tpu as pltpu

_B, _N = 8, 1024
_C = 64
_NUM_RBF = 32
_RH = 64
_L = 4
_K = 16
_CUTOFF = 5.0
_NCLS = 40


def _sigmoid(x):
    return 1.0 / (1.0 + jnp.exp(-x))


def _silu(x):
    return x * _sigmoid(x)


def _tfn_body(x_ref, emb_ref, rW1_ref, rb1_ref, rW2_ref, rb2_ref,
              Ws_ref, bs_ref, Wv_ref, Wg_ref, bg_ref,
              cW1_ref, cb1_ref, cW2_ref, cb2_ref, cW3_ref, cb3_ref,
              out_ref,
              P_ref, RU_ref, work_ref,
              s_ref, vx_ref, vy_ref, vz_ref,
              ms_ref, mvx_ref, mvy_ref, mvz_ref):
    f32 = jnp.float32
    x = x_ref[0]

    width = np.float32(_CUTOFF / (_NUM_RBF - 1))
    mu = jax.lax.broadcasted_iota(jnp.int32, (1, _NUM_RBF), 1).astype(f32) * width
    xn = jnp.sum(x * x, axis=1, keepdims=True)
    xxT = jax.lax.dot_general(x, x, (((1,), (1,)), ((), ())),
                              preferred_element_type=f32)
    d2 = xn + xn.T - 2.0 * xxT
    iota_j = jax.lax.broadcasted_iota(jnp.int32, (_N, _N), 1)
    iota_i = jax.lax.broadcasted_iota(jnp.int32, (_N, _N), 0)
    d2 = jnp.where(iota_i == iota_j, 1e9, d2)
    work_ref[...] = d2

    def _topk_body(k, _):
        work = work_ref[...]
        rmin = jnp.min(work, axis=1, keepdims=True)
        cand = jnp.where(work <= rmin, iota_j, jnp.int32(2**30))
        jmin = jnp.min(cand, axis=1, keepdims=True)
        onehot = (iota_j == jmin)
        work_ref[...] = jnp.where(onehot, 3e9, work)
        oh_f = jnp.where(onehot, 1.0, 0.0).astype(f32)
        P_ref[k] = oh_f.astype(jnp.bfloat16)
        xj = jnp.dot(oh_f, x, preferred_element_type=f32)
        rel = xj - x
        dist = jnp.sqrt(jnp.sum(rel * rel, axis=1, keepdims=True) + 1e-12)
        unit = rel / dist
        rbf = jnp.exp(-(((dist - mu) / width) ** 2))
        env = 0.5 * (jnp.cos(np.pi * jnp.clip(dist / _CUTOFF, 0.0, 1.0)) + 1.0)
        RU_ref[k, :, 0:_NUM_RBF] = rbf * env
        RU_ref[k, :, _NUM_RBF:_NUM_RBF + 3] = unit
        return 0

    jax.lax.fori_loop(0, _K, _topk_body, 0)

    s_ref[...] = jnp.broadcast_to(emb_ref[...], (_N, _C))
    vx_ref[...] = jnp.zeros((_N, _C), f32)
    vy_ref[...] = jnp.zeros((_N, _C), f32)
    vz_ref[...] = jnp.zeros((_N, _C), f32)

    def _run_edge_loop(l):
        table = jnp.concatenate(
            [s_ref[...], vx_ref[...], vy_ref[...], vz_ref[...]], axis=1)
        t_hi = table.astype(jnp.bfloat16)

        def _edge_one(k):
            Pk = P_ref[k]
            G = jnp.dot(Pk, t_hi, preferred_element_type=f32)
            R = RU_ref[k]
            rbf = R[:, 0:_NUM_RBF]
            ux = R[:, _NUM_RBF:_NUM_RBF + 1]
            uy = R[:, _NUM_RBF + 1:_NUM_RBF + 2]
            uz = R[:, _NUM_RBF + 2:_NUM_RBF + 3]

            h = _silu(jnp.dot(rbf, rW1_ref[l], preferred_element_type=f32)
                      + rb1_ref[l])
            w = jnp.dot(h, rW2_ref[l], preferred_element_type=f32) + rb2_ref[l]
            wss = w[:, 0:_C]
            wvs = w[:, _C:2 * _C]
            wsv = w[:, 2 * _C:3 * _C]
            wvv = w[:, 3 * _C:4 * _C]

            s_j = G[:, 0:_C]
            vjx = G[:, _C:2 * _C]
            vjy = G[:, 2 * _C:3 * _C]
            vjz = G[:, 3 * _C:4 * _C]
            vdotu = vjx * ux + vjy * uy + vjz * uz

            ms_ref[...] += wss * s_j + wvs * vdotu
            su = wsv * s_j
            mvx_ref[...] += wvv * vjx + su * ux
            mvy_ref[...] += wvv * vjy + su * uy
            mvz_ref[...] += wvv * vjz + su * uz

        def _edge_body(i, _):
            _edge_one(2 * i)
            _edge_one(2 * i + 1)
            return 0

        jax.lax.fori_loop(0, _K // 2, _edge_body, 0)

    for l in range(_L):
        ms_ref[...] = jnp.zeros((_N, _C), f32)
        mvx_ref[...] = jnp.zeros((_N, _C), f32)
        mvy_ref[...] = jnp.zeros((_N, _C), f32)
        mvz_ref[...] = jnp.zeros((_N, _C), f32)

        if l == 0:
            rW2l = jnp.concatenate(
                [rW2_ref[0][:, 0:_C], rW2_ref[0][:, 2 * _C:3 * _C]], axis=1)
            rb2l = jnp.concatenate(
                [rb2_ref[0][:, 0:_C], rb2_ref[0][:, 2 * _C:3 * _C]], axis=1)

            def _edge0_one(k):
                R = RU_ref[k]
                rbf = R[:, 0:_NUM_RBF]
                ux = R[:, _NUM_RBF:_NUM_RBF + 1]
                uy = R[:, _NUM_RBF + 1:_NUM_RBF + 2]
                uz = R[:, _NUM_RBF + 2:_NUM_RBF + 3]
                h = _silu(jnp.dot(rbf, rW1_ref[0], preferred_element_type=f32)
                          + rb1_ref[0])
                w = jnp.dot(h, rW2l, preferred_element_type=f32) + rb2l
                wss = w[:, 0:_C]
                wsv = w[:, _C:2 * _C]
                ms_ref[...] += wss
                mvx_ref[...] += wsv * ux
                mvy_ref[...] += wsv * uy
                mvz_ref[...] += wsv * uz

            def _edge0_body(i, _):
                _edge0_one(2 * i)
                _edge0_one(2 * i + 1)
                return 0

            jax.lax.fori_loop(0, _K // 2, _edge0_body, 0)
            emb_row = emb_ref[...]
            ms_ref[...] *= emb_row
            mvx_ref[...] *= emb_row
            mvy_ref[...] *= emb_row
            mvz_ref[...] *= emb_row
        else:
            _run_edge_loop(l)

        inv_k = f32(1.0 / _K)
        m_s = ms_ref[...] * inv_k
        s_cur = s_ref[...]
        s_upd = jnp.dot(m_s, Ws_ref[l], preferred_element_type=f32) + bs_ref[l]
        g = _sigmoid(jnp.dot(s_cur, Wg_ref[l], preferred_element_type=f32)
                     + bg_ref[l])
        gs = g[:, 0:_C]
        gv = g[:, _C:2 * _C]
        s_ref[...] = s_cur + gs * _silu(s_upd)
        Wvl = Wv_ref[l]
        vx_ref[...] += gv * jnp.dot(mvx_ref[...] * inv_k, Wvl,
                                    preferred_element_type=f32)
        vy_ref[...] += gv * jnp.dot(mvy_ref[...] * inv_k, Wvl,
                                    preferred_element_type=f32)
        vz_ref[...] += gv * jnp.dot(mvz_ref[...] * inv_k, Wvl,
                                    preferred_element_type=f32)

    vx, vy, vz = vx_ref[...], vy_ref[...], vz_ref[...]
    vn = jnp.sqrt(vx * vx + vy * vy + vz * vz + 1e-12)
    feat = jnp.concatenate([s_ref[...], vn], axis=1)
    pooled = jnp.mean(feat, axis=0, keepdims=True)
    h1 = jnp.maximum(jnp.dot(pooled, cW1_ref[...],
                             preferred_element_type=f32) + cb1_ref[...], 0.0)
    h2 = jnp.maximum(jnp.dot(h1, cW2_ref[...],
                             preferred_element_type=f32) + cb2_ref[...], 0.0)
    out_ref[0] = jnp.dot(h2, cW3_ref[...],
                         preferred_element_type=f32) + cb3_ref[...]


@jax.jit
def kernel(batch, emb, rW1, rb1, rW2, rb2, Ws, bs, Wv, Wg, bg,
           cW1, cb1, cW2, cb2, cW3, cb3):
    f32 = jnp.float32
    whole = lambda shape: pl.BlockSpec(shape, lambda b: (0,) * len(shape))
    in_specs = [
        pl.BlockSpec((1, _N, 3), lambda b: (b, 0, 0)),
        whole((1, _C)),
        whole((_L, _NUM_RBF, _RH)),
        whole((_L, 1, _RH)),
        whole((_L, _RH, 4 * _C)),
        whole((_L, 1, 4 * _C)),
        whole((_L, _C, _C)),
        whole((_L, 1, _C)),
        whole((_L, _C, _C)),
        whole((_L, _C, 2 * _C)),
        whole((_L, 1, 2 * _C)),
        whole((2 * _C, 128)),
        whole((1, 128)),
        whole((128, 64)),
        whole((1, 64)),
        whole((64, _NCLS)),
        whole((1, _NCLS)),
    ]
    out = pl.pallas_call(
        _tfn_body,
        grid=(_B,),
        in_specs=in_specs,
        out_specs=pl.BlockSpec((1, 1, _NCLS), lambda b: (b, 0, 0)),
        out_shape=jax.ShapeDtypeStruct((_B, 1, _NCLS), f32),
        scratch_shapes=[
            pltpu.VMEM((_K, _N, _N), jnp.bfloat16),
            pltpu.VMEM((_K, _N, 128), f32),
            pltpu.VMEM((_N, _N), f32),
            pltpu.VMEM((_N, _C), f32),
            pltpu.VMEM((_N, _C), f32),
            pltpu.VMEM((_N, _C), f32),
            pltpu.VMEM((_N, _C), f32),
            pltpu.VMEM((_N, _C), f32),
            pltpu.VMEM((_N, _C), f32),
            pltpu.VMEM((_N, _C), f32),
            pltpu.VMEM((_N, _C), f32),
        ],
    )(batch, emb.reshape(1, _C),
      rW1, rb1.reshape(_L, 1, _RH), rW2, rb2.reshape(_L, 1, 4 * _C),
      Ws, bs.reshape(_L, 1, _C), Wv, Wg, bg.reshape(_L, 1, 2 * _C),
      cW1, cb1.reshape(1, 128), cW2, cb2.reshape(1, 64),
      cW3, cb3.reshape(1, _NCLS))
    return out.reshape(_B, _NCLS)


# speedup vs baseline: 11.8383x; 1.0005x over previous
import functools
import numpy as np
import jax
import jax.numpy as jnp
from jax.experimental import pallas as pl
from jax.experimental.pallas import tpu as pltpu

_B, _N = 8, 1024
_C = 64
_NUM_RBF = 32
_RH = 64
_L = 4
_K = 16
_CUTOFF = 5.0
_NCLS = 40


def _sigmoid(x):
    return 1.0 / (1.0 + jnp.exp(-x))


def _silu(x):
    return x * _sigmoid(x)


def _tfn_body(x_ref, emb_ref, rW1_ref, rb1_ref, rW2_ref, rb2_ref,
              Ws_ref, bs_ref, Wv_ref, Wg_ref, bg_ref,
              cW1_ref, cb1_ref, cW2_ref, cb2_ref, cW3_ref, cb3_ref,
              out_ref,
              P_ref, RU_ref, work_ref,
              s_ref, vx_ref, vy_ref, vz_ref,
              ms_ref, mvx_ref, mvy_ref, mvz_ref):
    f32 = jnp.float32
    x = x_ref[0]

    width = np.float32(_CUTOFF / (_NUM_RBF - 1))
    mu = jax.lax.broadcasted_iota(jnp.int32, (1, _NUM_RBF), 1).astype(f32) * width
    xn = jnp.sum(x * x, axis=1, keepdims=True)
    xxT = jax.lax.dot_general(x, x, (((1,), (1,)), ((), ())),
                              preferred_element_type=f32)
    d2 = xn + xn.T - 2.0 * xxT
    iota_j = jax.lax.broadcasted_iota(jnp.int32, (_N, _N), 1)
    iota_i = jax.lax.broadcasted_iota(jnp.int32, (_N, _N), 0)
    d2 = jnp.where(iota_i == iota_j, 1e9, d2)
    work_ref[...] = d2

    def _topk_one(k):
        work = work_ref[...]
        rmin = jnp.min(work, axis=1, keepdims=True)
        cand = jnp.where(work <= rmin, iota_j, jnp.int32(2**30))
        jmin = jnp.min(cand, axis=1, keepdims=True)
        onehot = (iota_j == jmin)
        work_ref[...] = jnp.where(onehot, 3e9, work)
        oh_f = jnp.where(onehot, 1.0, 0.0).astype(f32)
        P_ref[k] = oh_f.astype(jnp.bfloat16)
        xj = jnp.dot(oh_f, x, preferred_element_type=f32)
        rel = xj - x
        dist = jnp.sqrt(jnp.sum(rel * rel, axis=1, keepdims=True) + 1e-12)
        unit = rel / dist
        rbf = jnp.exp(-(((dist - mu) / width) ** 2))
        env = 0.5 * (jnp.cos(np.pi * jnp.clip(dist / _CUTOFF, 0.0, 1.0)) + 1.0)
        RU_ref[k, :, 0:_NUM_RBF] = rbf * env
        RU_ref[k, :, _NUM_RBF:_NUM_RBF + 3] = unit

    def _topk_body(i, _):
        _topk_one(2 * i)
        _topk_one(2 * i + 1)
        return 0

    jax.lax.fori_loop(0, _K // 2, _topk_body, 0)

    s_ref[...] = jnp.broadcast_to(emb_ref[...], (_N, _C))
    vx_ref[...] = jnp.zeros((_N, _C), f32)
    vy_ref[...] = jnp.zeros((_N, _C), f32)
    vz_ref[...] = jnp.zeros((_N, _C), f32)

    def _run_edge_loop(l):
        table = jnp.concatenate(
            [s_ref[...], vx_ref[...], vy_ref[...], vz_ref[...]], axis=1)
        t_hi = table.astype(jnp.bfloat16)

        def _edge_one(k):
            Pk = P_ref[k]
            G = jnp.dot(Pk, t_hi, preferred_element_type=f32)
            R = RU_ref[k]
            rbf = R[:, 0:_NUM_RBF]
            ux = R[:, _NUM_RBF:_NUM_RBF + 1]
            uy = R[:, _NUM_RBF + 1:_NUM_RBF + 2]
            uz = R[:, _NUM_RBF + 2:_NUM_RBF + 3]

            h = _silu(jnp.dot(rbf, rW1_ref[l], preferred_element_type=f32)
                      + rb1_ref[l])
            w = jnp.dot(h, rW2_ref[l], preferred_element_type=f32) + rb2_ref[l]
            wss = w[:, 0:_C]
            wvs = w[:, _C:2 * _C]
            wsv = w[:, 2 * _C:3 * _C]
            wvv = w[:, 3 * _C:4 * _C]

            s_j = G[:, 0:_C]
            vjx = G[:, _C:2 * _C]
            vjy = G[:, 2 * _C:3 * _C]
            vjz = G[:, 3 * _C:4 * _C]
            vdotu = vjx * ux + vjy * uy + vjz * uz

            ms_ref[...] += wss * s_j + wvs * vdotu
            su = wsv * s_j
            mvx_ref[...] += wvv * vjx + su * ux
            mvy_ref[...] += wvv * vjy + su * uy
            mvz_ref[...] += wvv * vjz + su * uz

        def _edge_body(i, _):
            _edge_one(2 * i)
            _edge_one(2 * i + 1)
            return 0

        jax.lax.fori_loop(0, _K // 2, _edge_body, 0)

    for l in range(_L):
        ms_ref[...] = jnp.zeros((_N, _C), f32)
        mvx_ref[...] = jnp.zeros((_N, _C), f32)
        mvy_ref[...] = jnp.zeros((_N, _C), f32)
        mvz_ref[...] = jnp.zeros((_N, _C), f32)

        if l == 0:
            rW2l = jnp.concatenate(
                [rW2_ref[0][:, 0:_C], rW2_ref[0][:, 2 * _C:3 * _C]], axis=1)
            rb2l = jnp.concatenate(
                [rb2_ref[0][:, 0:_C], rb2_ref[0][:, 2 * _C:3 * _C]], axis=1)

            def _edge0_one(k):
                R = RU_ref[k]
                rbf = R[:, 0:_NUM_RBF]
                ux = R[:, _NUM_RBF:_NUM_RBF + 1]
                uy = R[:, _NUM_RBF + 1:_NUM_RBF + 2]
                uz = R[:, _NUM_RBF + 2:_NUM_RBF + 3]
                h = _silu(jnp.dot(rbf, rW1_ref[0], preferred_element_type=f32)
                          + rb1_ref[0])
                w = jnp.dot(h, rW2l, preferred_element_type=f32) + rb2l
                wss = w[:, 0:_C]
                wsv = w[:, _C:2 * _C]
                ms_ref[...] += wss
                mvx_ref[...] += wsv * ux
                mvy_ref[...] += wsv * uy
                mvz_ref[...] += wsv * uz

            def _edge0_body(i, _):
                _edge0_one(2 * i)
                _edge0_one(2 * i + 1)
                return 0

            jax.lax.fori_loop(0, _K // 2, _edge0_body, 0)
            emb_row = emb_ref[...]
            ms_ref[...] *= emb_row
            mvx_ref[...] *= emb_row
            mvy_ref[...] *= emb_row
            mvz_ref[...] *= emb_row
        else:
            _run_edge_loop(l)

        inv_k = f32(1.0 / _K)
        m_s = ms_ref[...] * inv_k
        s_cur = s_ref[...]
        s_upd = jnp.dot(m_s, Ws_ref[l], preferred_element_type=f32) + bs_ref[l]
        g = _sigmoid(jnp.dot(s_cur, Wg_ref[l], preferred_element_type=f32)
                     + bg_ref[l])
        gs = g[:, 0:_C]
        gv = g[:, _C:2 * _C]
        s_ref[...] = s_cur + gs * _silu(s_upd)
        Wvl = Wv_ref[l]
        vx_ref[...] += gv * jnp.dot(mvx_ref[...] * inv_k, Wvl,
                                    preferred_element_type=f32)
        vy_ref[...] += gv * jnp.dot(mvy_ref[...] * inv_k, Wvl,
                                    preferred_element_type=f32)
        vz_ref[...] += gv * jnp.dot(mvz_ref[...] * inv_k, Wvl,
                                    preferred_element_type=f32)

    vx, vy, vz = vx_ref[...], vy_ref[...], vz_ref[...]
    vn = jnp.sqrt(vx * vx + vy * vy + vz * vz + 1e-12)
    feat = jnp.concatenate([s_ref[...], vn], axis=1)
    pooled = jnp.mean(feat, axis=0, keepdims=True)
    h1 = jnp.maximum(jnp.dot(pooled, cW1_ref[...],
                             preferred_element_type=f32) + cb1_ref[...], 0.0)
    h2 = jnp.maximum(jnp.dot(h1, cW2_ref[...],
                             preferred_element_type=f32) + cb2_ref[...], 0.0)
    out_ref[0] = jnp.dot(h2, cW3_ref[...],
                         preferred_element_type=f32) + cb3_ref[...]


@jax.jit
def kernel(batch, emb, rW1, rb1, rW2, rb2, Ws, bs, Wv, Wg, bg,
           cW1, cb1, cW2, cb2, cW3, cb3):
    f32 = jnp.float32
    whole = lambda shape: pl.BlockSpec(shape, lambda b: (0,) * len(shape))
    in_specs = [
        pl.BlockSpec((1, _N, 3), lambda b: (b, 0, 0)),
        whole((1, _C)),
        whole((_L, _NUM_RBF, _RH)),
        whole((_L, 1, _RH)),
        whole((_L, _RH, 4 * _C)),
        whole((_L, 1, 4 * _C)),
        whole((_L, _C, _C)),
        whole((_L, 1, _C)),
        whole((_L, _C, _C)),
        whole((_L, _C, 2 * _C)),
        whole((_L, 1, 2 * _C)),
        whole((2 * _C, 128)),
        whole((1, 128)),
        whole((128, 64)),
        whole((1, 64)),
        whole((64, _NCLS)),
        whole((1, _NCLS)),
    ]
    out = pl.pallas_call(
        _tfn_body,
        grid=(_B,),
        in_specs=in_specs,
        out_specs=pl.BlockSpec((1, 1, _NCLS), lambda b: (b, 0, 0)),
        out_shape=jax.ShapeDtypeStruct((_B, 1, _NCLS), f32),
        scratch_shapes=[
            pltpu.VMEM((_K, _N, _N), jnp.bfloat16),
            pltpu.VMEM((_K, _N, 128), f32),
            pltpu.VMEM((_N, _N), f32),
            pltpu.VMEM((_N, _C), f32),
            pltpu.VMEM((_N, _C), f32),
            pltpu.VMEM((_N, _C), f32),
            pltpu.VMEM((_N, _C), f32),
            pltpu.VMEM((_N, _C), f32),
            pltpu.VMEM((_N, _C), f32),
            pltpu.VMEM((_N, _C), f32),
            pltpu.VMEM((_N, _C), f32),
        ],
    )(batch, emb.reshape(1, _C),
      rW1, rb1.reshape(_L, 1, _RH), rW2, rb2.reshape(_L, 1, 4 * _C),
      Ws, bs.reshape(_L, 1, _C), Wv, Wg, bg.reshape(_L, 1, 2 * _C),
      cW1, cb1.reshape(1, 128), cW2, cb2.reshape(1, 64),
      cW3, cb3.reshape(1, _NCLS))
    return out.reshape(_B, _NCLS)
